# baseline (speedup 1.0000x reference)
import jax
import jax.numpy as jnp
from jax.experimental import pallas as pl


_H = 32
_W = 32
_D = 256


def _body(ce_ref, re_ref, o_ref):
    ce = ce_ref[...]
    re = re_ref[...]
    ceT = ce.T
    reT = re.T
    o_ref[0, :_D] = jnp.broadcast_to(ceT[:, None, :], (_D, _H, _W))
    o_ref[0, _D:] = jnp.broadcast_to(reT[:, :, None], (_D, _H, _W))


def kernel(x, row_embed, col_embed):
    b, _, h, w = x.shape
    d = row_embed.shape[-1]
    ce = col_embed[:w]
    re = row_embed[:h]
    out = pl.pallas_call(
        _body,
        grid=(b,),
        in_specs=[
            pl.BlockSpec((w, d), lambda i: (0, 0)),
            pl.BlockSpec((h, d), lambda i: (0, 0)),
        ],
        out_specs=pl.BlockSpec((1, 2 * d, h, w), lambda i: (i, 0, 0, 0)),
        out_shape=jax.ShapeDtypeStruct((b, 2 * d, h, w), jnp.float32),
    )(ce, re)
    return out
